# baseline (speedup 1.0000x reference)
import functools

import jax
import jax.numpy as jnp
from jax import lax
from jax.experimental import pallas as pl
from jax.experimental.pallas import tpu as pltpu
from jax.experimental.pallas import tpu_sc as plsc

GRID = 128
B = 1048576
NC = 2
NS = 16
NW = NC * NS
B_PER_W = B // NW
CHUNK = 2048
NSUB = B_PER_W // CHUNK
VPC = CHUNK // 16


def _flat_index(ox, oy, oz, vx, vy, vz):
    s = jnp.maximum(jnp.maximum(jnp.abs(vx), jnp.abs(vy)), jnp.abs(vz))
    a = vx / s
    b = vy / s
    c = vz / s
    one = jnp.float32(1.0)
    face = jnp.where(
        c <= -one, jnp.int32(5),
        jnp.where(c >= one, jnp.int32(4),
                  jnp.where(b <= -one, jnp.int32(3),
                            jnp.where(b >= one, jnp.int32(2),
                                      jnp.where(a <= -one, jnp.int32(1),
                                                jnp.int32(0))))))

    def coord(o):
        w = (o / jnp.float32(2.0) + jnp.float32(0.5)) * jnp.float32(GRID - 1)
        w = jnp.minimum(jnp.maximum(w, jnp.float32(0.0)), jnp.float32(GRID - 1))
        return w.astype(jnp.int32)

    i = coord(ox)
    j = coord(oy)
    k = coord(oz)
    return ((i * GRID + j) * GRID + k) * 6 + face


def _sc_body(rays_hbm, numer_hbm, denom_hbm, out_hbm,
             ox_b, oy_b, oz_b, vx_b, vy_b, vz_b,
             idx_buf, n_buf, d_buf, ob_buf, sem):
    wid = lax.axis_index("s") * NC + lax.axis_index("c")
    base = wid * B_PER_W

    for sub in range(NSUB):
        cbase = base + sub * CHUNK
        for comp, buf in enumerate((ox_b, oy_b, oz_b, vx_b, vy_b, vz_b)):
            pltpu.sync_copy(rays_hbm.at[pl.ds(comp * B + cbase, CHUNK)], buf)

        def compute(t, _):
            sl = pl.ds(t * 16, 16)
            idx_buf[sl] = _flat_index(ox_b[sl], oy_b[sl], oz_b[sl],
                                      vx_b[sl], vy_b[sl], vz_b[sl])
            return 0

        lax.fori_loop(0, VPC, compute, 0)

        cp_n = pltpu.async_copy(numer_hbm.at[idx_buf], n_buf, sem)
        cp_d = pltpu.async_copy(denom_hbm.at[idx_buf], d_buf, sem)
        cp_n.wait()
        cp_d.wait()

        def divide(t, _):
            sl = pl.ds(t * 16, 16)
            n = n_buf[sl].astype(jnp.float32)
            d = d_buf[sl].astype(jnp.float32)
            ob_buf[sl] = n / d
            return 0

        lax.fori_loop(0, VPC, divide, 0)

        pltpu.sync_copy(ob_buf, out_hbm.at[pl.ds(cbase, CHUNK)])


@jax.jit
def _run(rays_flat, numer_flat, denom_flat):
    mesh = plsc.VectorSubcoreMesh(core_axis_name="c", subcore_axis_name="s")
    call = functools.partial(
        pl.kernel,
        mesh=mesh,
        out_type=jax.ShapeDtypeStruct((B,), jnp.float32),
        scratch_types=[
            pltpu.VMEM((CHUNK,), jnp.float32),
            pltpu.VMEM((CHUNK,), jnp.float32),
            pltpu.VMEM((CHUNK,), jnp.float32),
            pltpu.VMEM((CHUNK,), jnp.float32),
            pltpu.VMEM((CHUNK,), jnp.float32),
            pltpu.VMEM((CHUNK,), jnp.float32),
            pltpu.VMEM((CHUNK,), jnp.int32),
            pltpu.VMEM((CHUNK,), jnp.int32),
            pltpu.VMEM((CHUNK,), jnp.int32),
            pltpu.VMEM((CHUNK,), jnp.float32),
            pltpu.SemaphoreType.DMA,
        ],
    )(_sc_body)
    return call(rays_flat, numer_flat, denom_flat)


def kernel(norm_ray_origins, viewdirs, numer, denom):
    rays = jnp.concatenate([norm_ray_origins, viewdirs], axis=1)
    rays_flat = rays.T.reshape(-1)
    numer_flat = numer.reshape(-1)
    denom_flat = denom.reshape(-1)
    return _run(rays_flat, numer_flat, denom_flat)


# speedup vs baseline: 13.1361x; 13.1361x over previous
import functools

import jax
import jax.numpy as jnp
from jax import lax
from jax.experimental import pallas as pl
from jax.experimental.pallas import tpu as pltpu
from jax.experimental.pallas import tpu_sc as plsc

GRID = 128
B = 1048576
NC = 2
NS = 16
NW = NC * NS
B_PER_W = B // NW
CHUNK = 4096
NSUB = B_PER_W // CHUNK
VPC = CHUNK // 16


def _flat_index(ox, oy, oz, vx, vy, vz):
    s = jnp.maximum(jnp.maximum(jnp.abs(vx), jnp.abs(vy)), jnp.abs(vz))
    a = vx / s
    b = vy / s
    c = vz / s
    one = jnp.float32(1.0)
    face = jnp.where(
        c <= -one, jnp.int32(5),
        jnp.where(c >= one, jnp.int32(4),
                  jnp.where(b <= -one, jnp.int32(3),
                            jnp.where(b >= one, jnp.int32(2),
                                      jnp.where(a <= -one, jnp.int32(1),
                                                jnp.int32(0))))))

    def coord(o):
        w = (o / jnp.float32(2.0) + jnp.float32(0.5)) * jnp.float32(GRID - 1)
        w = jnp.minimum(jnp.maximum(w, jnp.float32(0.0)), jnp.float32(GRID - 1))
        return w.astype(jnp.int32)

    i = coord(ox)
    j = coord(oy)
    k = coord(oz)
    return (i * 6 + face) * 16384 + j * 128 + k


def _sc_body(o_hbm, v_hbm, numer_hbm, denom_hbm, out_hbm,
             c00, c01, c02, c03, c04, c05,
             c10, c11, c12, c13, c14, c15,
             idx0, idx1, n0, d0, n1, d1, ob0, ob1,
             in_sem, g_sem, out_sem):
    wid = lax.axis_index("s") * NC + lax.axis_index("c")
    base = wid * B_PER_W
    comp = ((c00, c01, c02, c03, c04, c05),
            (c10, c11, c12, c13, c14, c15))
    idxb = (idx0, idx1)
    ndb = ((n0, d0), (n1, d1))
    obb = (ob0, ob1)

    def fire_in(s):
        cbase = base + s * CHUNK
        cps = []
        for c in range(3):
            cps.append(pltpu.async_copy(
                o_hbm.at[pl.ds(c * B + cbase, CHUNK)],
                comp[s % 2][c], in_sem))
            cps.append(pltpu.async_copy(
                v_hbm.at[pl.ds(c * B + cbase, CHUNK)],
                comp[s % 2][c + 3], in_sem))
        return cps

    def fire_gather(s):
        p = s % 2
        cp_n = pltpu.async_copy(numer_hbm.at[idxb[p]], ndb[p][0], g_sem)
        cp_d = pltpu.async_copy(denom_hbm.at[idxb[p]], ndb[p][1], g_sem)
        return [cp_n, cp_d]

    def compute(s):
        p = s % 2
        cbuf = comp[p]
        ibuf = idxb[p]

        def body(t, _):
            sl = pl.ds(t * 16, 16)
            ibuf[sl] = _flat_index(cbuf[0][sl], cbuf[1][sl], cbuf[2][sl],
                                   cbuf[3][sl], cbuf[4][sl], cbuf[5][sl])
            return 0

        lax.fori_loop(0, VPC, body, 0)

    def divide(s):
        p = s % 2
        nd = ndb[p]
        ob = obb[p]

        def body(t, _):
            sl = pl.ds(t * 16, 16)
            n = nd[0][sl].astype(jnp.float32)
            d = nd[1][sl].astype(jnp.float32)
            ob[sl] = n / d
            return 0

        lax.fori_loop(0, VPC, body, 0)

    def fire_out(s):
        p = s % 2
        return [pltpu.async_copy(
            obb[p], out_hbm.at[pl.ds(base + s * CHUNK, CHUNK)], out_sem)]

    pend_in = fire_in(0)
    pend_g = []
    pend_out = []
    for s in range(NSUB):
        for cp in pend_in:
            cp.wait()
        pend_in = []
        compute(s)
        if s > 0:
            for cp in pend_g:
                cp.wait()
            pend_g = []
        new_g = fire_gather(s)
        if s + 1 < NSUB:
            pend_in = fire_in(s + 1)
        if s > 0:
            if len(pend_out) == 2:
                for cp in pend_out.pop(0):
                    cp.wait()
            divide(s - 1)
            pend_out.append(fire_out(s - 1))
        pend_g = new_g
    for cp in pend_g:
        cp.wait()
    divide(NSUB - 1)
    pend_out.append(fire_out(NSUB - 1))
    for cps in pend_out:
        for cp in cps:
            cp.wait()


@jax.jit
def _run(o_flat, v_flat, numer_flat, denom_flat):
    mesh = plsc.VectorSubcoreMesh(core_axis_name="c", subcore_axis_name="s")
    call = functools.partial(
        pl.kernel,
        mesh=mesh,
        out_type=jax.ShapeDtypeStruct((B,), jnp.float32),
        scratch_types=(
            [pltpu.VMEM((CHUNK,), jnp.float32)] * 12
            + [pltpu.VMEM((CHUNK,), jnp.int32)] * 2
            + [pltpu.VMEM((CHUNK,), jnp.int32)] * 4
            + [pltpu.VMEM((CHUNK,), jnp.float32)] * 2
            + [pltpu.SemaphoreType.DMA] * 3
        ),
    )(_sc_body)
    return call(o_flat, v_flat, numer_flat, denom_flat)


def kernel(norm_ray_origins, viewdirs, numer, denom):
    o_flat = norm_ray_origins.T.reshape(-1)
    v_flat = viewdirs.T.reshape(-1)
    numer_flat = jnp.transpose(numer, (0, 3, 1, 2)).reshape(-1)
    denom_flat = jnp.transpose(denom, (0, 3, 1, 2)).reshape(-1)
    return _run(o_flat, v_flat, numer_flat, denom_flat)
